# baseline (speedup 1.0000x reference)
import functools

import jax
import jax.numpy as jnp
from jax import lax
from jax.experimental import pallas as pl
from jax.experimental.pallas import tpu as pltpu
from jax.experimental.pallas import tpu_sc as plsc

N = 100000
E = 3200000
F_IN = 16
H = 32
G = 64
E_AUX = 200000

HALF = N // 2
TRASH = HALF

EG = 25088
E_PAD = EG * 128
ROWS_PER_TILE_P = EG // 16
ROWS_PER_TILE_H = EG // 32

AG = 1568
EA_PAD = AG * 128
AUX_ROWS_PER_TILE = AG // 32

_MESH = plsc.VectorSubcoreMesh(core_axis_name="core", subcore_axis_name="subcore")
_CP = pltpu.CompilerParams(needs_layout_passes=False, use_tc_tiling_on_sc=False)

def _sc_histogram(dst_r):
    @pl.kernel(
        out_type=jax.ShapeDtypeStruct((32, 6250, 16), jnp.float32),
        mesh=_MESH, compiler_params=_CP,
        scratch_types=[pltpu.VMEM((6251, 16), jnp.float32),
                       pltpu.VMEM((16, 128), jnp.int32)],
    )
    def k(dst_hbm, o_hbm, hist, dstv):
        c = lax.axis_index("core")
        s = lax.axis_index("subcore")
        w = c * 16 + s

        @pl.loop(0, 6251)
        def _(r):
            hist[r, :] = jnp.zeros((16,), jnp.float32)

        @pl.loop(0, ROWS_PER_TILE_H, step=16)
        def _(t):
            pltpu.sync_copy(dst_hbm.at[pl.ds(w * ROWS_PER_TILE_H + t, 16)], dstv)

            @pl.loop(0, 16)
            def _(g):
                @pl.loop(0, 8)
                def _(j):
                    dv = dstv[g, pl.ds(j * 16, 16)]
                    rv = jnp.minimum(lax.shift_right_logical(dv, 4), 6250)
                    cv = lax.bitwise_and(dv, 15)
                    plsc.addupdate_scatter(hist, [rv, cv], jnp.ones((16,), jnp.float32))

        pltpu.sync_copy(hist.at[pl.ds(0, 6250)], o_hbm.at[w])

    return k(dst_r)


def _sc_propagate(src_r, dst_r, hs):
    @pl.kernel(
        out_type=jax.ShapeDtypeStruct((N, H), jnp.float32),
        mesh=_MESH, compiler_params=_CP,
        scratch_types=[pltpu.VMEM((16, 128), jnp.int32),
                       pltpu.VMEM((16, 128), jnp.int32),
                       pltpu.VMEM((16, 128), jnp.int32),
                       pltpu.VMEM((128, H), jnp.float32),
                       pltpu.VMEM((128, H), jnp.float32),
                       pltpu.VMEM_SHARED((HALF + 16, H), jnp.float32),
                       pltpu.SemaphoreType.DMA],
    )
    def k(src_hbm, dst_hbm, hs_hbm, p_hbm, srcv, dstv, locv, rows, zb, acc, sem):
        c = lax.axis_index("core")
        s = lax.axis_index("subcore")
        base = c * HALF

        @pl.loop(0, 128)
        def _(i):
            zb[i, pl.ds(0, 16)] = jnp.zeros((16,), jnp.float32)
            zb[i, pl.ds(16, 16)] = jnp.zeros((16,), jnp.float32)

        @pl.loop(0, 24)
        def _(r):
            pltpu.sync_copy(zb, acc.at[pl.ds(s * 3126 + r * 128, 128)])
        pltpu.sync_copy(zb.at[pl.ds(0, 54)], acc.at[pl.ds(s * 3126 + 3072, 54)])
        plsc.subcore_barrier()

        @pl.loop(0, ROWS_PER_TILE_P, step=16)
        def _(t):
            row0 = s * ROWS_PER_TILE_P + t
            pltpu.sync_copy(src_hbm.at[pl.ds(row0, 16)], srcv)
            pltpu.sync_copy(dst_hbm.at[pl.ds(row0, 16)], dstv)

            @pl.loop(0, 16)
            def _(g):
                @pl.loop(0, 8)
                def _(j):
                    dv = dstv[g, pl.ds(j * 16, 16)]
                    lv = dv - base
                    okm = jnp.logical_and(lv >= 0, lv < HALF)
                    locv[g, pl.ds(j * 16, 16)] = jnp.where(okm, lv, TRASH)

                pltpu.sync_copy(hs_hbm.at[srcv.at[g]], rows)
                pltpu.sync_copy(rows, acc.at[locv.at[g]], add=True)

        plsc.subcore_barrier()
        pltpu.sync_copy(acc.at[pl.ds(s * 3125, 3125)],
                        p_hbm.at[pl.ds(base + s * 3125, 3125)])

    return k(src_r, dst_r, hs)


def _sc_aux_gather(a0_r, a1_r, zt, ht):
    @pl.kernel(
        out_type=[jax.ShapeDtypeStruct((EA_PAD, H), jnp.float32),
                  jax.ShapeDtypeStruct((EA_PAD, H), jnp.float32)],
        mesh=_MESH, compiler_params=_CP,
        scratch_types=[pltpu.VMEM((AUX_ROWS_PER_TILE, 128), jnp.int32),
                       pltpu.VMEM((AUX_ROWS_PER_TILE, 128), jnp.int32),
                       pltpu.VMEM((128, H), jnp.float32),
                       pltpu.VMEM((128, H), jnp.float32)],
    )
    def k(a0_hbm, a1_hbm, z_hbm, h_hbm, za_hbm, hb_hbm, a0v, a1v, zr, hr):
        c = lax.axis_index("core")
        s = lax.axis_index("subcore")
        w = c * 16 + s
        r0 = w * AUX_ROWS_PER_TILE
        pltpu.sync_copy(a0_hbm.at[pl.ds(r0, AUX_ROWS_PER_TILE)], a0v)
        pltpu.sync_copy(a1_hbm.at[pl.ds(r0, AUX_ROWS_PER_TILE)], a1v)

        @pl.loop(0, AUX_ROWS_PER_TILE)
        def _(g):
            e0 = (r0 + g) * 128
            pltpu.sync_copy(z_hbm.at[a0v.at[g]], zr)
            pltpu.sync_copy(zr, za_hbm.at[pl.ds(e0, 128)])
            pltpu.sync_copy(h_hbm.at[a1v.at[g]], hr)
            pltpu.sync_copy(hr, hb_hbm.at[pl.ds(e0, 128)])

    return k(a0_r, a1_r, zt, ht)


_AB = EA_PAD // 32


def _tc_aux(za, hb, bb):
    def body(a_ref, b_ref, bias_ref, o_ref):
        o_ref[...] = jnp.sum(a_ref[...] * b_ref[...], axis=1,
                             keepdims=True) + bias_ref[...]

    return pl.pallas_call(
        body,
        grid=(32,),
        in_specs=[pl.BlockSpec((_AB, H), lambda i: (i, 0)),
                  pl.BlockSpec((_AB, H), lambda i: (i, 0)),
                  pl.BlockSpec((1, 1), lambda i: (0, 0))],
        out_specs=pl.BlockSpec((_AB, 1), lambda i: (i, 0)),
        out_shape=jax.ShapeDtypeStruct((EA_PAD, 1), jnp.float32),
    )(za, hb, bb)


_RB = 5000
_NB = N // _RB


def _tc_dinv(parts):
    def body(p_ref, o_ref):
        deg = jnp.sum(p_ref[...], axis=0, keepdims=True) + 1.0
        o_ref[...] = lax.rsqrt(jnp.maximum(deg, 1.0))

    return pl.pallas_call(
        body,
        out_shape=jax.ShapeDtypeStruct((1, N), jnp.float32),
    )(parts)


def _tc_stage2(x, W1, dinv_col):
    def body(x_ref, w_ref, d_ref, o_ref):
        h = jnp.dot(x_ref[...], w_ref[...], preferred_element_type=jnp.float32)
        o_ref[...] = h * d_ref[...]

    return pl.pallas_call(
        body,
        grid=(_NB,),
        in_specs=[pl.BlockSpec((_RB, F_IN), lambda i: (i, 0)),
                  pl.BlockSpec((F_IN, H), lambda i: (0, 0)),
                  pl.BlockSpec((_RB, 1), lambda i: (i, 0))],
        out_specs=pl.BlockSpec((_RB, H), lambda i: (i, 0)),
        out_shape=jax.ShapeDtypeStruct((N, H), jnp.float32),
    )(x, W1, dinv_col)


def _tc_stage4(p1, hs1, dinv_col, b1, W2):
    def body(p_ref, hs_ref, d_ref, b_ref, w_ref, o_ref):
        d = d_ref[...]
        t = (p_ref[...] + hs_ref[...]) * d + b_ref[...]
        t = jnp.maximum(t, 0.0)
        o_ref[...] = jnp.dot(t, w_ref[...],
                             preferred_element_type=jnp.float32) * d

    return pl.pallas_call(
        body,
        grid=(_NB,),
        in_specs=[pl.BlockSpec((_RB, H), lambda i: (i, 0)),
                  pl.BlockSpec((_RB, H), lambda i: (i, 0)),
                  pl.BlockSpec((_RB, 1), lambda i: (i, 0)),
                  pl.BlockSpec((1, H), lambda i: (0, 0)),
                  pl.BlockSpec((H, H), lambda i: (0, 0))],
        out_specs=pl.BlockSpec((_RB, H), lambda i: (i, 0)),
        out_shape=jax.ShapeDtypeStruct((N, H), jnp.float32),
    )(p1, hs1, dinv_col, b1, W2)


def _tc_stage6(p2, hs2, dinv_col, b2, batch_col, Wr, br, Wb0):
    def body(p_ref, hs_ref, d_ref, b_ref, bat_ref, wr_ref, br_ref, wb_ref,
             h_ref, z_ref, reg_ref, sums, cnt):
        i = pl.program_id(0)
        h = (p_ref[...] + hs_ref[...]) * d_ref[...] + b_ref[...]
        h_ref[...] = h
        z_ref[...] = jnp.dot(h, wb_ref[...], preferred_element_type=jnp.float32)

        onehot = (bat_ref[...] ==
                  lax.broadcasted_iota(jnp.int32, (_RB, G), 1)).astype(jnp.float32)
        psum = lax.dot_general(onehot, h, (((0,), (0,)), ((), ())),
                               preferred_element_type=jnp.float32)
        pcnt = lax.dot_general(onehot, jnp.ones((_RB, 1), jnp.float32),
                               (((0,), (0,)), ((), ())),
                               preferred_element_type=jnp.float32)

        @pl.when(i == 0)
        def _():
            sums[...] = jnp.zeros_like(sums)
            cnt[...] = jnp.zeros_like(cnt)

        sums[...] += psum
        cnt[...] += pcnt

        @pl.when(i == _NB - 1)
        def _():
            pooled = sums[...] / jnp.maximum(cnt[...], 1.0)
            reg_ref[...] = jnp.dot(pooled, wr_ref[...],
                                   preferred_element_type=jnp.float32) + br_ref[...]

    return pl.pallas_call(
        body,
        grid=(_NB,),
        in_specs=[pl.BlockSpec((_RB, H), lambda i: (i, 0)),
                  pl.BlockSpec((_RB, H), lambda i: (i, 0)),
                  pl.BlockSpec((_RB, 1), lambda i: (i, 0)),
                  pl.BlockSpec((1, H), lambda i: (0, 0)),
                  pl.BlockSpec((_RB, 1), lambda i: (i, 0)),
                  pl.BlockSpec((H, 1), lambda i: (0, 0)),
                  pl.BlockSpec((1, 1), lambda i: (0, 0)),
                  pl.BlockSpec((H, H), lambda i: (0, 0))],
        out_specs=[pl.BlockSpec((_RB, H), lambda i: (i, 0)),
                   pl.BlockSpec((_RB, H), lambda i: (i, 0)),
                   pl.BlockSpec((G, 1), lambda i: (0, 0))],
        out_shape=[jax.ShapeDtypeStruct((N, H), jnp.float32),
                   jax.ShapeDtypeStruct((N, H), jnp.float32),
                   jax.ShapeDtypeStruct((G, 1), jnp.float32)],
        scratch_shapes=[pltpu.VMEM((G, H), jnp.float32),
                        pltpu.VMEM((G, 1), jnp.float32)],
    )(p2, hs2, dinv_col, b2, batch_col, Wr, br, Wb0)


def kernel(x, edge_index, batch, edge_index_aux, W1, b1, W2, b2, Wr, br, Wb, bb):
    src = edge_index[0]
    dst = edge_index[1]
    pad = E_PAD - E
    src_r = jnp.concatenate(
        [src, jnp.zeros((pad,), jnp.int32)]).reshape(EG, 128)
    dst_r = jnp.concatenate(
        [dst, jnp.full((pad,), 2 * N, jnp.int32)]).reshape(EG, 128)

    apad = EA_PAD - E_AUX
    a0_r = jnp.concatenate(
        [edge_index_aux[0], jnp.zeros((apad,), jnp.int32)]).reshape(AG, 128)
    a1_r = jnp.concatenate(
        [edge_index_aux[1], jnp.zeros((apad,), jnp.int32)]).reshape(AG, 128)

    parts = _sc_histogram(dst_r).reshape(32, N)
    dinv_col = _tc_dinv(parts).reshape(N, 1)

    hs1 = _tc_stage2(x, W1, dinv_col)
    p1 = _sc_propagate(src_r, dst_r, hs1)
    hs2 = _tc_stage4(p1, hs1, dinv_col, b1.reshape(1, H), W2)
    p2 = _sc_propagate(src_r, dst_r, hs2)
    h, z, reg_output = _tc_stage6(p2, hs2, dinv_col, b2.reshape(1, H),
                                  batch.reshape(N, 1), Wr, br.reshape(1, 1),
                                  Wb[0])

    za, hb = _sc_aux_gather(a0_r, a1_r, z, h)
    edge_pred = _tc_aux(za, hb, bb.reshape(1, 1))[:E_AUX]
    return (reg_output, edge_pred)


# speedup vs baseline: 1.0345x; 1.0345x over previous
import functools

import jax
import jax.numpy as jnp
from jax import lax
from jax.experimental import pallas as pl
from jax.experimental.pallas import tpu as pltpu
from jax.experimental.pallas import tpu_sc as plsc

N = 100000
E = 3200000
F_IN = 16
H = 32
G = 64
E_AUX = 200000

HALF = N // 2
TRASH = HALF

EG = 25088
E_PAD = EG * 128
ROWS_PER_TILE_P = EG // 16
ROWS_PER_TILE_H = EG // 32

AG = 1568
EA_PAD = AG * 128
AUX_ROWS_PER_TILE = AG // 32

_MESH = plsc.VectorSubcoreMesh(core_axis_name="core", subcore_axis_name="subcore")
_CP = pltpu.CompilerParams(needs_layout_passes=False, use_tc_tiling_on_sc=False)

def _sc_histogram(dst_r):
    @pl.kernel(
        out_type=jax.ShapeDtypeStruct((32, 6250, 16), jnp.float32),
        mesh=_MESH, compiler_params=_CP,
        scratch_types=[pltpu.VMEM((6251, 16), jnp.float32),
                       pltpu.VMEM((16, 128), jnp.int32)],
    )
    def k(dst_hbm, o_hbm, hist, dstv):
        c = lax.axis_index("core")
        s = lax.axis_index("subcore")
        w = c * 16 + s

        @pl.loop(0, 6251)
        def _(r):
            hist[r, :] = jnp.zeros((16,), jnp.float32)

        @pl.loop(0, ROWS_PER_TILE_H, step=16)
        def _(t):
            pltpu.sync_copy(dst_hbm.at[pl.ds(w * ROWS_PER_TILE_H + t, 16)], dstv)

            @pl.loop(0, 16)
            def _(g):
                @pl.loop(0, 8)
                def _(j):
                    dv = dstv[g, pl.ds(j * 16, 16)]
                    rv = jnp.minimum(lax.shift_right_logical(dv, 4), 6250)
                    cv = lax.bitwise_and(dv, 15)
                    plsc.addupdate_scatter(hist, [rv, cv], jnp.ones((16,), jnp.float32))

        pltpu.sync_copy(hist.at[pl.ds(0, 6250)], o_hbm.at[w])

    return k(dst_r)


def _sc_propagate(src_r, dst_r, hs):
    @pl.kernel(
        out_type=jax.ShapeDtypeStruct((N, H), jnp.float32),
        mesh=_MESH, compiler_params=_CP,
        scratch_types=[pltpu.VMEM((16, 128), jnp.int32),
                       pltpu.VMEM((16, 128), jnp.int32),
                       pltpu.VMEM((16, 128), jnp.int32),
                       pltpu.VMEM((128, H), jnp.float32),
                       pltpu.VMEM((128, H), jnp.float32),
                       pltpu.VMEM_SHARED((HALF + 16, H), jnp.float32),
                       pltpu.SemaphoreType.DMA],
    )
    def k(src_hbm, dst_hbm, hs_hbm, p_hbm, srcv, dstv, locv, rows, zb, acc, sem):
        c = lax.axis_index("core")
        s = lax.axis_index("subcore")
        base = c * HALF

        @pl.loop(0, 128)
        def _(i):
            zb[i, pl.ds(0, 16)] = jnp.zeros((16,), jnp.float32)
            zb[i, pl.ds(16, 16)] = jnp.zeros((16,), jnp.float32)

        @pl.loop(0, 24)
        def _(r):
            pltpu.sync_copy(zb, acc.at[pl.ds(s * 3126 + r * 128, 128)])
        pltpu.sync_copy(zb.at[pl.ds(0, 54)], acc.at[pl.ds(s * 3126 + 3072, 54)])
        plsc.subcore_barrier()

        @pl.loop(0, ROWS_PER_TILE_P, step=16)
        def _(t):
            row0 = s * ROWS_PER_TILE_P + t
            pltpu.sync_copy(src_hbm.at[pl.ds(row0, 16)], srcv)
            pltpu.sync_copy(dst_hbm.at[pl.ds(row0, 16)], dstv)

            @pl.loop(0, 16)
            def _(g):
                @pl.loop(0, 8)
                def _(j):
                    dv = dstv[g, pl.ds(j * 16, 16)]
                    lv = dv - base
                    okm = jnp.logical_and(lv >= 0, lv < HALF)
                    locv[g, pl.ds(j * 16, 16)] = jnp.where(okm, lv, TRASH)

                pltpu.sync_copy(hs_hbm.at[srcv.at[g]], rows)
                pltpu.sync_copy(rows, acc.at[locv.at[g]], add=True)

        plsc.subcore_barrier()
        pltpu.sync_copy(acc.at[pl.ds(s * 3125, 3125)],
                        p_hbm.at[pl.ds(base + s * 3125, 3125)])

    return k(src_r, dst_r, hs)


def _sc_aux(a0_r, a1_r, zt, ht, bb):
    @pl.kernel(
        out_type=jax.ShapeDtypeStruct((AG, 128), jnp.float32),
        mesh=_MESH, compiler_params=_CP,
        scratch_types=[pltpu.VMEM((AUX_ROWS_PER_TILE, 128), jnp.int32),
                       pltpu.VMEM((AUX_ROWS_PER_TILE, 128), jnp.int32),
                       pltpu.VMEM((128, H), jnp.float32),
                       pltpu.VMEM((128, H), jnp.float32),
                       pltpu.VMEM((AUX_ROWS_PER_TILE, 128), jnp.float32),
                       pltpu.VMEM((16,), jnp.float32)],
    )
    def k(a0_hbm, a1_hbm, z_hbm, h_hbm, bb_hbm, o_hbm, a0v, a1v, zr, hr, ob, bbv):
        c = lax.axis_index("core")
        s = lax.axis_index("subcore")
        w = c * 16 + s
        r0 = w * AUX_ROWS_PER_TILE
        pltpu.sync_copy(a0_hbm.at[pl.ds(r0, AUX_ROWS_PER_TILE)], a0v)
        pltpu.sync_copy(a1_hbm.at[pl.ds(r0, AUX_ROWS_PER_TILE)], a1v)
        pltpu.sync_copy(bb_hbm, bbv)

        @pl.loop(0, AUX_ROWS_PER_TILE)
        def _(g):
            pltpu.sync_copy(z_hbm.at[a0v.at[g]], zr)
            pltpu.sync_copy(h_hbm.at[a1v.at[g]], hr)

            @pl.loop(0, 8)
            def _(kk):
                rowv = lax.broadcasted_iota(jnp.int32, (16,), 0) + kk * 16
                acc = bbv[...]
                for j in range(H):
                    colv = jnp.full((16,), j, jnp.int32)
                    va = plsc.load_gather(zr, [rowv, colv])
                    vb = plsc.load_gather(hr, [rowv, colv])
                    acc = acc + va * vb
                ob[g, pl.ds(kk * 16, 16)] = acc

        pltpu.sync_copy(ob, o_hbm.at[pl.ds(r0, AUX_ROWS_PER_TILE)])

    return k(a0_r, a1_r, zt, ht, bb)


_RB = 5000
_NB = N // _RB


def _tc_dinv(parts):
    def body(p_ref, o_ref):
        deg = jnp.sum(p_ref[...], axis=0, keepdims=True) + 1.0
        o_ref[...] = lax.rsqrt(jnp.maximum(deg, 1.0))

    return pl.pallas_call(
        body,
        out_shape=jax.ShapeDtypeStruct((1, N), jnp.float32),
    )(parts)


def _tc_stage2(x, W1, dinv_col):
    def body(x_ref, w_ref, d_ref, o_ref):
        h = jnp.dot(x_ref[...], w_ref[...], preferred_element_type=jnp.float32)
        o_ref[...] = h * d_ref[...]

    return pl.pallas_call(
        body,
        grid=(_NB,),
        in_specs=[pl.BlockSpec((_RB, F_IN), lambda i: (i, 0)),
                  pl.BlockSpec((F_IN, H), lambda i: (0, 0)),
                  pl.BlockSpec((_RB, 1), lambda i: (i, 0))],
        out_specs=pl.BlockSpec((_RB, H), lambda i: (i, 0)),
        out_shape=jax.ShapeDtypeStruct((N, H), jnp.float32),
    )(x, W1, dinv_col)


def _tc_stage4(p1, hs1, dinv_col, b1, W2):
    def body(p_ref, hs_ref, d_ref, b_ref, w_ref, o_ref):
        d = d_ref[...]
        t = (p_ref[...] + hs_ref[...]) * d + b_ref[...]
        t = jnp.maximum(t, 0.0)
        o_ref[...] = jnp.dot(t, w_ref[...],
                             preferred_element_type=jnp.float32) * d

    return pl.pallas_call(
        body,
        grid=(_NB,),
        in_specs=[pl.BlockSpec((_RB, H), lambda i: (i, 0)),
                  pl.BlockSpec((_RB, H), lambda i: (i, 0)),
                  pl.BlockSpec((_RB, 1), lambda i: (i, 0)),
                  pl.BlockSpec((1, H), lambda i: (0, 0)),
                  pl.BlockSpec((H, H), lambda i: (0, 0))],
        out_specs=pl.BlockSpec((_RB, H), lambda i: (i, 0)),
        out_shape=jax.ShapeDtypeStruct((N, H), jnp.float32),
    )(p1, hs1, dinv_col, b1, W2)


def _tc_stage6(p2, hs2, dinv_col, b2, batch_col, Wr, br, Wb0):
    def body(p_ref, hs_ref, d_ref, b_ref, bat_ref, wr_ref, br_ref, wb_ref,
             h_ref, z_ref, reg_ref, sums, cnt):
        i = pl.program_id(0)
        h = (p_ref[...] + hs_ref[...]) * d_ref[...] + b_ref[...]
        h_ref[...] = h
        z_ref[...] = jnp.dot(h, wb_ref[...], preferred_element_type=jnp.float32)

        onehot = (bat_ref[...] ==
                  lax.broadcasted_iota(jnp.int32, (_RB, G), 1)).astype(jnp.float32)
        psum = lax.dot_general(onehot, h, (((0,), (0,)), ((), ())),
                               preferred_element_type=jnp.float32)
        pcnt = lax.dot_general(onehot, jnp.ones((_RB, 1), jnp.float32),
                               (((0,), (0,)), ((), ())),
                               preferred_element_type=jnp.float32)

        @pl.when(i == 0)
        def _():
            sums[...] = jnp.zeros_like(sums)
            cnt[...] = jnp.zeros_like(cnt)

        sums[...] += psum
        cnt[...] += pcnt

        @pl.when(i == _NB - 1)
        def _():
            pooled = sums[...] / jnp.maximum(cnt[...], 1.0)
            reg_ref[...] = jnp.dot(pooled, wr_ref[...],
                                   preferred_element_type=jnp.float32) + br_ref[...]

    return pl.pallas_call(
        body,
        grid=(_NB,),
        in_specs=[pl.BlockSpec((_RB, H), lambda i: (i, 0)),
                  pl.BlockSpec((_RB, H), lambda i: (i, 0)),
                  pl.BlockSpec((_RB, 1), lambda i: (i, 0)),
                  pl.BlockSpec((1, H), lambda i: (0, 0)),
                  pl.BlockSpec((_RB, 1), lambda i: (i, 0)),
                  pl.BlockSpec((H, 1), lambda i: (0, 0)),
                  pl.BlockSpec((1, 1), lambda i: (0, 0)),
                  pl.BlockSpec((H, H), lambda i: (0, 0))],
        out_specs=[pl.BlockSpec((_RB, H), lambda i: (i, 0)),
                   pl.BlockSpec((_RB, H), lambda i: (i, 0)),
                   pl.BlockSpec((G, 1), lambda i: (0, 0))],
        out_shape=[jax.ShapeDtypeStruct((N, H), jnp.float32),
                   jax.ShapeDtypeStruct((N, H), jnp.float32),
                   jax.ShapeDtypeStruct((G, 1), jnp.float32)],
        scratch_shapes=[pltpu.VMEM((G, H), jnp.float32),
                        pltpu.VMEM((G, 1), jnp.float32)],
    )(p2, hs2, dinv_col, b2, batch_col, Wr, br, Wb0)


def kernel(x, edge_index, batch, edge_index_aux, W1, b1, W2, b2, Wr, br, Wb, bb):
    src = edge_index[0]
    dst = edge_index[1]
    pad = E_PAD - E
    src_r = jnp.concatenate(
        [src, jnp.zeros((pad,), jnp.int32)]).reshape(EG, 128)
    dst_r = jnp.concatenate(
        [dst, jnp.full((pad,), 2 * N, jnp.int32)]).reshape(EG, 128)

    apad = EA_PAD - E_AUX
    a0_r = jnp.concatenate(
        [edge_index_aux[0], jnp.zeros((apad,), jnp.int32)]).reshape(AG, 128)
    a1_r = jnp.concatenate(
        [edge_index_aux[1], jnp.zeros((apad,), jnp.int32)]).reshape(AG, 128)

    parts = _sc_histogram(dst_r).reshape(32, N)
    dinv_col = _tc_dinv(parts).reshape(N, 1)

    hs1 = _tc_stage2(x, W1, dinv_col)
    p1 = _sc_propagate(src_r, dst_r, hs1)
    hs2 = _tc_stage4(p1, hs1, dinv_col, b1.reshape(1, H), W2)
    p2 = _sc_propagate(src_r, dst_r, hs2)
    h, z, reg_output = _tc_stage6(p2, hs2, dinv_col, b2.reshape(1, H),
                                  batch.reshape(N, 1), Wr, br.reshape(1, 1),
                                  Wb[0])

    ep = _sc_aux(a0_r, a1_r, z, h, jnp.broadcast_to(bb, (16,)))
    edge_pred = ep.reshape(EA_PAD, 1)[:E_AUX]
    return (reg_output, edge_pred)
